# baseline (speedup 1.0000x reference)
import functools

import jax
import jax.numpy as jnp
from jax import lax
from jax.experimental import pallas as pl
from jax.experimental.pallas import tpu as pltpu
from jax.experimental.pallas import tpu_sc as plsc

_MESH = plsc.VectorSubcoreMesh(core_axis_name="c", subcore_axis_name="s")
_NW = 32
_PARAMS = pltpu.CompilerParams(
    use_tc_tiling_on_sc=False, needs_layout_passes=False)


def _transpose_call(v, d):
    n_chunks = 1953
    n_pairs = 31

    @functools.partial(
        pl.kernel, mesh=_MESH,
        out_type=jax.ShapeDtypeStruct((v, d), jnp.float32),
        compiler_params=_PARAMS,
        scratch_types=[
            pltpu.VMEM((2, 4, 4, 8, 128), jnp.float32),
            pltpu.VMEM((2, 512, 32), jnp.float32),
            pltpu.SemaphoreType.DMA,
            pltpu.SemaphoreType.DMA,
            pltpu.SemaphoreType.DMA,
            pltpu.SemaphoreType.DMA,
        ],
    )
    def tk(vr4_hbm, out_hbm, src_v, dst_v, s0, s1, w0, w1):
        wid = lax.axis_index("s") * 2 + lax.axis_index("c")
        ssem = (s0, s1)
        wsem = (w0, w1)
        iota = lax.iota(jnp.int32, 16)
        viota = [iota + vb * 16 for vb in range(8)]

        def load_chunk(b, c):
            for te in range(4):
                pltpu.async_copy(
                    vr4_hbm.at[te, pl.ds(c * 4, 4)], src_v.at[b, te], ssem[b])

        def drain_loads(b):
            for te in range(4):
                pltpu.make_async_copy(
                    vr4_hbm.at[0, pl.ds(0, 4)], src_v.at[b, te],
                    ssem[b]).wait()

        def shuffle(b):
            @plsc.parallel_loop(0, 128, unroll=16)
            def _(g):
                e = g & 31
                q = g >> 5
                e_splat = jnp.full((16,), 1, jnp.int32) * e
                dq = dst_v.at[b, pl.ds(q * 128, 128)]
                for vb in range(8):
                    vals = src_v[b, e >> 3, q, e & 7, pl.ds(vb * 16, 16)]
                    plsc.store_scatter(dq, [viota[vb], e_splat], vals)

        def store_chunk(b, c):
            pltpu.async_copy(
                dst_v.at[b], out_hbm.at[pl.ds(c * 512, 512)], wsem[b])

        def drain_store(b):
            pltpu.make_async_copy(
                dst_v.at[b], out_hbm.at[pl.ds(0, 512)], wsem[b]).wait()

        store_chunk(0, wid)
        store_chunk(1, wid + 32)

        def body(k, carry):
            for b in range(2):
                c = wid + 32 * (2 * k + b)

                @pl.when(c < n_chunks)
                def _():
                    drain_store(b)
                    load_chunk(b, c)

            for b in range(2):
                c = wid + 32 * (2 * k + b)

                @pl.when(c < n_chunks)
                def _():
                    drain_loads(b)
                    shuffle(b)
                    store_chunk(b, c)

            return carry

        lax.fori_loop(0, n_pairs, body, 0)
        drain_store(0)
        drain_store(1)

        @pl.when(wid == 0)
        def _():
            for te in range(4):
                pltpu.async_copy(
                    vr4_hbm.at[te, 7812, :, pl.ds(0, 64)],
                    src_v.at[0, te, 0, :, pl.ds(0, 64)], ssem[0])
            for te in range(4):
                pltpu.make_async_copy(
                    vr4_hbm.at[0, 0, :, pl.ds(0, 64)],
                    src_v.at[0, te, 0, :, pl.ds(0, 64)], ssem[0]).wait()

            @plsc.parallel_loop(0, 32, unroll=4)
            def _(e):
                e_splat = jnp.full((16,), 1, jnp.int32) * e
                for vb in range(4):
                    vals = src_v[0, e >> 3, 0, e & 7, pl.ds(vb * 16, 16)]
                    plsc.store_scatter(
                        dst_v.at[0], [viota[vb], e_splat], vals)

            pltpu.async_copy(
                dst_v.at[0, pl.ds(0, 64)],
                out_hbm.at[pl.ds(999936, 64)], wsem[0])
            pltpu.make_async_copy(
                dst_v.at[0, pl.ds(0, 64)],
                out_hbm.at[pl.ds(0, 64)], wsem[0]).wait()

    return tk


def _gather_call(n_total, hist, d):
    n_pairs = 25

    @functools.partial(
        pl.kernel, mesh=_MESH,
        out_type=jax.ShapeDtypeStruct((hist, 4, 32, 8, 128), jnp.float32),
        compiler_params=_PARAMS,
        scratch_types=[
            pltpu.VMEM((2, 512), jnp.int32),
            pltpu.VMEM((2, 512, 32), jnp.float32),
            pltpu.VMEM((2, 4, 4, 8, 128), jnp.float32),
            pltpu.SemaphoreType.DMA,
            pltpu.SemaphoreType.DMA,
            pltpu.SemaphoreType.DMA,
            pltpu.SemaphoreType.DMA,
            pltpu.SemaphoreType.DMA,
            pltpu.SemaphoreType.DMA,
        ],
    )
    def gk(idx_hbm, table_hbm, out_hbm, idx_v, rows_v, obuf_v,
           i0, i1, g0, g1, w0, w1):
        wid = lax.axis_index("s") * 2 + lax.axis_index("c")
        isem = (i0, i1)
        gsem = (g0, g1)
        wsem = (w0, w1)
        iota = lax.iota(jnp.int32, 16)
        viota = [iota + bb * 16 for bb in range(8)]
        base = wid * (hist * 128)

        def load_idx(b, u):
            pltpu.async_copy(
                idx_hbm.at[pl.ds(pl.multiple_of(base + u * 512, 8), 512)],
                idx_v.at[b], isem[b])

        def drain_idx(b):
            pltpu.make_async_copy(
                idx_hbm.at[pl.ds(0, 512)], idx_v.at[b], isem[b]).wait()

        def shuffle(b):
            @plsc.parallel_loop(0, 128, unroll=16)
            def _(gg):
                e = gg & 31
                g = gg >> 5
                e_splat = jnp.full((16,), 1, jnp.int32) * e
                rg = rows_v.at[b, pl.ds(g * 128, 128)]
                for bb in range(8):
                    vals = plsc.load_gather(rg, [viota[bb], e_splat])
                    obuf_v[b, g, e >> 3, e & 7, pl.ds(bb * 16, 16)] = vals

        def store_tiles(b, u):
            for g in range(4):
                for te in range(4):
                    pltpu.async_copy(
                        obuf_v.at[b, g, te],
                        out_hbm.at[u * 4 + g, te, wid], wsem[b])

        def drain_store(b):
            for _ in range(16):
                pltpu.make_async_copy(
                    obuf_v.at[b, 0, 0], out_hbm.at[0, 0, 0], wsem[b]).wait()

        load_idx(0, 0)
        load_idx(1, 1)
        store_tiles(0, 0)
        store_tiles(1, 1)

        def body(k, carry):
            gathers = []
            for b in range(2):
                drain_store(b)
                drain_idx(b)
                gathers.append(pltpu.async_copy(
                    table_hbm.at[idx_v.at[b]], rows_v.at[b], gsem[b]))
            for b in range(2):
                u = 2 * k + b
                gathers[b].wait()
                load_idx(b, jnp.minimum(u + 2, 2 * n_pairs - 1))
                shuffle(b)
                store_tiles(b, u)
            return carry

        lax.fori_loop(0, n_pairs, body, 0)
        drain_store(0)
        drain_store(1)
        drain_idx(0)
        drain_idx(1)

    return gk


def kernel(x, table):
    b, h = x.shape
    v, d = table.shape
    n = b * h
    tp = jnp.pad(table, ((0, 64), (0, 0)))
    vr4 = tp.reshape(7813, 128, 4, 8).transpose(2, 0, 3, 1)
    t_rm = _transpose_call(v, d)(vr4)
    xq = x.T.reshape(h, 32, 128).transpose(1, 0, 2).reshape(n)
    out5 = _gather_call(n, h, d)(xq, t_rm)
    return out5.transpose(2, 4, 0, 1, 3).reshape(b, h, d)


# speedup vs baseline: 1.1597x; 1.1597x over previous
import functools

import jax
import jax.numpy as jnp
from jax import lax
from jax.experimental import pallas as pl
from jax.experimental.pallas import tpu as pltpu
from jax.experimental.pallas import tpu_sc as plsc

_MESH = plsc.VectorSubcoreMesh(core_axis_name="c", subcore_axis_name="s")
_NW = 32
_PARAMS = pltpu.CompilerParams(
    use_tc_tiling_on_sc=False, needs_layout_passes=False)


def _transpose_call(v, d):
    n_chunks = 1953
    n_pairs = 31

    @functools.partial(
        pl.kernel, mesh=_MESH,
        out_type=jax.ShapeDtypeStruct((v, d), jnp.float32),
        compiler_params=_PARAMS,
        scratch_types=[
            pltpu.VMEM((2, 4, 4, 8, 128), jnp.float32),
            pltpu.VMEM((2, 512, 32), jnp.float32),
            pltpu.SemaphoreType.DMA,
            pltpu.SemaphoreType.DMA,
            pltpu.SemaphoreType.DMA,
            pltpu.SemaphoreType.DMA,
        ],
    )
    def tk(vr4_hbm, out_hbm, src_v, dst_v, s0, s1, w0, w1):
        wid = lax.axis_index("s") * 2 + lax.axis_index("c")
        ssem = (s0, s1)
        wsem = (w0, w1)
        iota = lax.iota(jnp.int32, 16)
        viota = [iota + vb * 16 for vb in range(8)]

        def load_chunk(b, c):
            for te in range(4):
                pltpu.async_copy(
                    vr4_hbm.at[te, pl.ds(c * 4, 4)], src_v.at[b, te], ssem[b])

        def drain_loads(b):
            for te in range(4):
                pltpu.make_async_copy(
                    vr4_hbm.at[0, pl.ds(0, 4)], src_v.at[b, te],
                    ssem[b]).wait()

        def shuffle(b):
            @plsc.parallel_loop(0, 128, unroll=16)
            def _(g):
                e = g & 31
                q = g >> 5
                e_splat = jnp.full((16,), 1, jnp.int32) * e
                dq = dst_v.at[b, pl.ds(q * 128, 128)]
                for vb in range(8):
                    vals = src_v[b, e >> 3, q, e & 7, pl.ds(vb * 16, 16)]
                    plsc.store_scatter(dq, [viota[vb], e_splat], vals)

        def store_chunk(b, c):
            pltpu.async_copy(
                dst_v.at[b], out_hbm.at[pl.ds(c * 512, 512)], wsem[b])

        def drain_store(b):
            pltpu.make_async_copy(
                dst_v.at[b], out_hbm.at[pl.ds(0, 512)], wsem[b]).wait()

        store_chunk(0, wid)
        store_chunk(1, wid + 32)

        def body(k, carry):
            for b in range(2):
                c = wid + 32 * (2 * k + b)

                @pl.when(c < n_chunks)
                def _():
                    drain_store(b)
                    load_chunk(b, c)

            for b in range(2):
                c = wid + 32 * (2 * k + b)

                @pl.when(c < n_chunks)
                def _():
                    drain_loads(b)
                    shuffle(b)
                    store_chunk(b, c)

            return carry

        lax.fori_loop(0, n_pairs, body, 0)
        drain_store(0)
        drain_store(1)

        @pl.when(wid == 0)
        def _():
            for te in range(4):
                pltpu.async_copy(
                    vr4_hbm.at[te, 7812, :, pl.ds(0, 64)],
                    src_v.at[0, te, 0, :, pl.ds(0, 64)], ssem[0])
            for te in range(4):
                pltpu.make_async_copy(
                    vr4_hbm.at[0, 0, :, pl.ds(0, 64)],
                    src_v.at[0, te, 0, :, pl.ds(0, 64)], ssem[0]).wait()

            @plsc.parallel_loop(0, 32, unroll=4)
            def _(e):
                e_splat = jnp.full((16,), 1, jnp.int32) * e
                for vb in range(4):
                    vals = src_v[0, e >> 3, 0, e & 7, pl.ds(vb * 16, 16)]
                    plsc.store_scatter(
                        dst_v.at[0], [viota[vb], e_splat], vals)

            pltpu.async_copy(
                dst_v.at[0, pl.ds(0, 64)],
                out_hbm.at[pl.ds(999936, 64)], wsem[0])
            pltpu.make_async_copy(
                dst_v.at[0, pl.ds(0, 64)],
                out_hbm.at[pl.ds(0, 64)], wsem[0]).wait()

    return tk


def _gather_call(n_total, hist, d):
    n_pairs = 25

    @functools.partial(
        pl.kernel, mesh=_MESH,
        out_type=jax.ShapeDtypeStruct((hist, 4, 32, 8, 128), jnp.float32),
        compiler_params=_PARAMS,
        scratch_types=[
            pltpu.VMEM((2, 512), jnp.int32),
            pltpu.VMEM((2, 512, 32), jnp.float32),
            pltpu.VMEM((2, 4, 4, 8, 128), jnp.float32),
            pltpu.SemaphoreType.DMA,
            pltpu.SemaphoreType.DMA,
            pltpu.SemaphoreType.DMA,
            pltpu.SemaphoreType.DMA,
            pltpu.SemaphoreType.DMA,
            pltpu.SemaphoreType.DMA,
        ],
    )
    def gk(idx_hbm, table_hbm, out_hbm, idx_v, rows_v, obuf_v,
           i0, i1, g0, g1, w0, w1):
        wid = lax.axis_index("s") * 2 + lax.axis_index("c")
        isem = (i0, i1)
        gsem = (g0, g1)
        wsem = (w0, w1)
        iota = lax.iota(jnp.int32, 16)
        viota = [iota + bb * 16 for bb in range(8)]
        base = wid * (hist * 128)

        def load_idx(b, u):
            pltpu.async_copy(
                idx_hbm.at[pl.ds(pl.multiple_of(base + u * 512, 8), 512)],
                idx_v.at[b], isem[b])

        def drain_idx(b):
            pltpu.make_async_copy(
                idx_hbm.at[pl.ds(0, 512)], idx_v.at[b], isem[b]).wait()

        def shuffle(b):
            @plsc.parallel_loop(0, 32, unroll=4)
            def _(e):
                e_splat = jnp.full((16,), 1, jnp.int32) * e
                for g in range(4):
                    rg = rows_v.at[b, pl.ds(g * 128, 128)]
                    for bb in range(8):
                        vals = plsc.load_gather(rg, [viota[bb], e_splat])
                        obuf_v[b, g, e >> 3, e & 7, pl.ds(bb * 16, 16)] = vals

        def store_tiles(b, u):
            for g in range(4):
                for te in range(4):
                    pltpu.async_copy(
                        obuf_v.at[b, g, te],
                        out_hbm.at[u * 4 + g, te, wid], wsem[b])

        def drain_store(b):
            for _ in range(16):
                pltpu.make_async_copy(
                    obuf_v.at[b, 0, 0], out_hbm.at[0, 0, 0], wsem[b]).wait()

        load_idx(0, 0)
        load_idx(1, 1)
        store_tiles(0, 0)
        store_tiles(1, 1)

        def body(k, carry):
            gathers = []
            for b in range(2):
                drain_store(b)
                drain_idx(b)
                gathers.append(pltpu.async_copy(
                    table_hbm.at[idx_v.at[b]], rows_v.at[b], gsem[b]))
            for b in range(2):
                u = 2 * k + b
                gathers[b].wait()
                load_idx(b, jnp.minimum(u + 2, 2 * n_pairs - 1))
                shuffle(b)
                store_tiles(b, u)
            return carry

        lax.fori_loop(0, n_pairs, body, 0)
        drain_store(0)
        drain_store(1)
        drain_idx(0)
        drain_idx(1)

    return gk


def kernel(x, table):
    b, h = x.shape
    v, d = table.shape
    n = b * h
    xq = x.T.reshape(h, 32, 128).transpose(1, 0, 2).reshape(n)
    out5 = _gather_call(n, h, d)(xq, table)
    return out5.transpose(2, 4, 0, 1, 3).reshape(b, h, d)
